# baseline (speedup 1.0000x reference)
import functools

import jax
import jax.numpy as jnp
from jax import lax
from jax.experimental import pallas as pl
from jax.experimental.pallas import tpu as pltpu
from jax.experimental.pallas import tpu_sc as plsc

_FEATURES = 32


@functools.lru_cache(maxsize=None)
def _make_lookup(B, D, n_workers, chunk):
    b_per_w = B // n_workers
    n_chunks = b_per_w // chunk
    mesh = plsc.VectorSubcoreMesh(core_axis_name="c", subcore_axis_name="s")

    @functools.partial(
        pl.kernel,
        mesh=mesh,
        out_type=jax.ShapeDtypeStruct((B, D), jnp.float32),
        scratch_types=[
            pltpu.VMEM((2, chunk), jnp.int32),
            pltpu.VMEM((2, chunk, D), jnp.float32),
            pltpu.SemaphoreType.DMA,
            pltpu.SemaphoreType.DMA,
        ],
        compiler_params=pltpu.CompilerParams(use_tc_tiling_on_sc=False),
    )
    def lookup(idx_hbm, table_hbm, out_hbm, idx_v, rows_v, sem_g, sem_o):
        wid = lax.axis_index("s") * 2 + lax.axis_index("c")
        base = wid * b_per_w

        nsub = 4
        sub = chunk // nsub

        def start_gather(slot):
            return [
                pltpu.async_copy(
                    table_hbm.at[idx_v.at[slot, pl.ds(q * sub, sub)]],
                    rows_v.at[slot, pl.ds(q * sub, sub)],
                    sem_g,
                )
                for q in range(nsub)
            ]

        pltpu.sync_copy(idx_hbm.at[pl.ds(base, chunk)], idx_v.at[0])
        gathers = [start_gather(0)]
        outs = [None, None]
        for g in range(n_chunks):
            s = g % 2
            ns = (g + 1) % 2
            if g + 1 < n_chunks:
                pltpu.sync_copy(
                    idx_hbm.at[pl.ds(base + (g + 1) * chunk, chunk)],
                    idx_v.at[ns],
                )
            for cp in gathers[g]:
                cp.wait()
            if g + 1 < n_chunks:
                if outs[ns] is not None:
                    outs[ns].wait()
                gathers.append(start_gather(ns))
            outs[s] = pltpu.async_copy(
                rows_v.at[s], out_hbm.at[pl.ds(base + g * chunk, chunk)], sem_o
            )
        if n_chunks >= 2:
            outs[(n_chunks - 2) % 2].wait()
        outs[(n_chunks - 1) % 2].wait()

    return lookup


def kernel(inputs, embedding):
    B = inputs.shape[0] * inputs.shape[1]
    flat_idx = inputs.reshape(B).astype(jnp.int32)
    out = _make_lookup(B, _FEATURES, 32, 1664)(flat_idx, embedding)
    return out.reshape(inputs.shape + (_FEATURES,))


# speedup vs baseline: 1.1297x; 1.1297x over previous
import functools

import jax
import jax.numpy as jnp
from jax import lax
from jax.experimental import pallas as pl
from jax.experimental.pallas import tpu as pltpu
from jax.experimental.pallas import tpu_sc as plsc

_F = 32
_BI = 8192


def _tc_transpose_body(xt_ref, out_ref):
    y = xt_ref[...].T
    y3 = y.reshape(_BI // 4, 4, _F)
    out_ref[...] = jnp.concatenate([y3[:, k, :] for k in range(4)], axis=1)


@functools.lru_cache(maxsize=None)
def _make_tc_transpose(V):
    n_blocks = pl.cdiv(V, _BI)
    return pl.pallas_call(
        _tc_transpose_body,
        grid=(n_blocks,),
        in_specs=[pl.BlockSpec((_F, _BI), lambda i: (0, i))],
        out_specs=pl.BlockSpec((_BI // 4, 128), lambda i: (i, 0)),
        out_shape=jax.ShapeDtypeStruct((V * _F // 128, 128), jnp.float32),
    )


@functools.lru_cache(maxsize=None)
def _make_lookup(B, n_workers, chunk, V):
    b_per_w = B // n_workers
    n_chunks = b_per_w // chunk
    mesh = plsc.VectorSubcoreMesh(core_axis_name="c", subcore_axis_name="s")

    @functools.partial(
        pl.kernel,
        mesh=mesh,
        out_type=jax.ShapeDtypeStruct((B, _F), jnp.float32),
        scratch_types=[
            pltpu.VMEM((2, chunk), jnp.int32),
            pltpu.VMEM((2, chunk, _F), jnp.float32),
            pltpu.SemaphoreType.DMA,
            pltpu.SemaphoreType.DMA,
        ],
        compiler_params=pltpu.CompilerParams(use_tc_tiling_on_sc=False),
    )
    def lookup(idx_hbm, table_hbm, out_hbm, idx_v, rows_v, sem_g, sem_o):
        wid = lax.axis_index("s") * 2 + lax.axis_index("c")
        base = wid * b_per_w

        def start_gather(slot):
            return pltpu.async_copy(
                table_hbm.at[idx_v.at[slot]], rows_v.at[slot], sem_g
            )

        pltpu.sync_copy(idx_hbm.at[pl.ds(base, chunk)], idx_v.at[0])
        gathers = [start_gather(0)]
        outs = [None, None]
        for g in range(n_chunks):
            s = g % 2
            ns = (g + 1) % 2
            if g + 1 < n_chunks:
                pltpu.sync_copy(
                    idx_hbm.at[pl.ds(base + (g + 1) * chunk, chunk)],
                    idx_v.at[ns],
                )
            gathers[g].wait()
            if g + 1 < n_chunks:
                if outs[ns] is not None:
                    outs[ns].wait()
                gathers.append(start_gather(ns))
            outs[s] = pltpu.async_copy(
                rows_v.at[s], out_hbm.at[pl.ds(base + g * chunk, chunk)], sem_o
            )
        if n_chunks >= 2:
            outs[(n_chunks - 2) % 2].wait()
        outs[(n_chunks - 1) % 2].wait()

    return lookup


def kernel(inputs, embedding):
    V, F = embedding.shape
    B = inputs.shape[0] * inputs.shape[1]
    flat_idx = inputs.reshape(B)
    table_lin = _make_tc_transpose(V)(embedding.T)
    table_rm = table_lin.reshape(V, F)
    out = _make_lookup(B, 32, 1664, V)(flat_idx, table_rm)
    return out.reshape(inputs.shape + (F,))


# speedup vs baseline: 1.5695x; 1.3893x over previous
import functools

import jax
import jax.numpy as jnp
from jax import lax
from jax.experimental import pallas as pl
from jax.experimental.pallas import tpu as pltpu
from jax.experimental.pallas import tpu_sc as plsc

_F = 32
_BI = 8192
_Q = _BI // 4


def _tc_relayout_body(xt_ref, out_ref):
    x = xt_ref[...]
    fi = lax.broadcasted_iota(jnp.int32, (_F, 128), 0)
    li = lax.broadcasted_iota(jnp.int32, (_F, 128), 1)
    acc = None
    for k in range(4):
        ek = jnp.where(li == fi + _F * k, 1.0, 0.0)
        part = lax.dot_general(
            x[:, _Q * k:_Q * (k + 1)], ek,
            (((0,), (0,)), ((), ())),
            preferred_element_type=jnp.float32,
        )
        acc = part if acc is None else acc + part
    out_ref[...] = acc


@functools.lru_cache(maxsize=None)
def _make_tc_relayout(V):
    n_blocks = pl.cdiv(V, _BI)
    return pl.pallas_call(
        _tc_relayout_body,
        grid=(n_blocks,),
        in_specs=[pl.BlockSpec((_F, _BI), lambda i: (0, i))],
        out_specs=pl.BlockSpec((_Q, 128), lambda i: (i, 0)),
        out_shape=jax.ShapeDtypeStruct((n_blocks * _Q, 128), jnp.float32),
    )


@functools.lru_cache(maxsize=None)
def _make_lookup(B, n_workers, chunk, table_rows):
    b_per_w = B // n_workers
    n_chunks = b_per_w // chunk
    mesh = plsc.VectorSubcoreMesh(core_axis_name="c", subcore_axis_name="s")

    @functools.partial(
        pl.kernel,
        mesh=mesh,
        out_type=jax.ShapeDtypeStruct((B, _F), jnp.float32),
        scratch_types=[
            pltpu.VMEM((2, chunk), jnp.int32),
            pltpu.VMEM((2, chunk, _F), jnp.float32),
            pltpu.SemaphoreType.DMA,
            pltpu.SemaphoreType.DMA,
        ],
        compiler_params=pltpu.CompilerParams(use_tc_tiling_on_sc=False),
    )
    def lookup(idx_hbm, table_hbm, out_hbm, idx_v, rows_v, sem_g, sem_o):
        wid = lax.axis_index("s") * 2 + lax.axis_index("c")
        base = wid * b_per_w

        def remap(slot):
            ref = idx_v.at[slot]
            for b in range(chunk // 16):
                sl = pl.ds(b * 16, 16)
                t = ref[sl]
                j = (
                    ((t >> 13) << 13)
                    + ((t & 2047) << 2)
                    + ((t >> 11) & 3)
                )
                ref[sl] = j

        def start_gather(slot):
            return pltpu.async_copy(
                table_hbm.at[idx_v.at[slot]], rows_v.at[slot], sem_g
            )

        pltpu.sync_copy(idx_hbm.at[pl.ds(base, chunk)], idx_v.at[0])
        remap(0)
        gathers = [start_gather(0)]
        outs = [None, None]
        for g in range(n_chunks):
            s = g % 2
            ns = (g + 1) % 2
            if g + 1 < n_chunks:
                pltpu.sync_copy(
                    idx_hbm.at[pl.ds(base + (g + 1) * chunk, chunk)],
                    idx_v.at[ns],
                )
                remap(ns)
            gathers[g].wait()
            if g + 1 < n_chunks:
                if outs[ns] is not None:
                    outs[ns].wait()
                gathers.append(start_gather(ns))
            outs[s] = pltpu.async_copy(
                rows_v.at[s], out_hbm.at[pl.ds(base + g * chunk, chunk)], sem_o
            )
        if n_chunks >= 2:
            outs[(n_chunks - 2) % 2].wait()
        outs[(n_chunks - 1) % 2].wait()

    return lookup


def kernel(inputs, embedding):
    V, F = embedding.shape
    B = inputs.shape[0] * inputs.shape[1]
    flat_idx = inputs.reshape(B)
    table_lin = _make_tc_relayout(V)(embedding.T)
    table_rows = table_lin.shape[0] * (128 // F)
    table32 = table_lin.reshape(table_rows, F)
    out = _make_lookup(B, 32, 1664, table_rows)(flat_idx, table32)
    return out.reshape(inputs.shape + (F,))
